# baseline (speedup 1.0000x reference)
import functools

import jax
import jax.numpy as jnp
from jax import lax
from jax.experimental import pallas as pl
from jax.experimental.pallas import tpu as pltpu
from jax.experimental.pallas import tpu_sc as plsc

N, E, F, H, C = 10000, 320000, 128, 128, 64
NP = 10240
CHUNK = 64
EP = 327680
NC, NS = 2, 16
CHUNKS_TOTAL = EP // CHUNK
CHUNKS_PER_CORE = CHUNKS_TOTAL // NC
CHUNKS_PER_TILE = CHUNKS_PER_CORE // NS
ROWS_PER_TILE = NP // NS
SECT = 8
CH0, CH1 = 256, 64
NC0_CHUNKS = NS * CH0
NBUF = 3


def _sc_scatter(table, src2d, dst2d):
    mesh = plsc.VectorSubcoreMesh(core_axis_name="c", subcore_axis_name="s")

    @functools.partial(
        pl.kernel,
        out_type=jax.ShapeDtypeStruct((NC, NP, F), jnp.float32),
        mesh=mesh,
        scratch_types=[
            pltpu.VMEM_SHARED((NP, F), jnp.float32),
            pltpu.VMEM((2, SECT, CHUNK), jnp.int32),
            pltpu.VMEM((2, SECT, CHUNK), jnp.int32),
            pltpu.VMEM((NBUF, CHUNK, F), jnp.float32),
        ] + [pltpu.SemaphoreType.DMA] * (2 * NBUF + 2),
    )
    def k(table_hbm, src_hbm, dst_hbm, out_hbm,
          agg_sh, src_v, dst_v, rows_v, *sems):
        gsem = sems[:NBUF]
        ssem = sems[NBUF:2 * NBUF]
        isem = sems[2 * NBUF]
        bsem = sems[2 * NBUF + 1]
        c = lax.axis_index("c")
        s = lax.axis_index("s")
        row0 = s * ROWS_PER_TILE
        nseg = ROWS_PER_TILE // CHUNK

        def _idx_copies(chunk0, t):
            return (
                pltpu.make_async_copy(src_hbm.at[pl.ds(chunk0, SECT)],
                                      src_v.at[t], isem),
                pltpu.make_async_copy(dst_hbm.at[pl.ds(chunk0, SECT)],
                                      dst_v.at[t], isem),
            )

        def _slot(j):
            return (j // SECT) % 2, j % SECT

        def _wait_gather(j, b):
            t, l = _slot(j)
            pltpu.make_async_copy(table_hbm.at[src_v.at[t, l]],
                                  rows_v.at[b], gsem[b]).wait()

        def _start_scatter(j, b):
            t, l = _slot(j)
            pltpu.async_copy(rows_v.at[b], agg_sh.at[dst_v.at[t, l]],
                             ssem[b], add=True)

        def _wait_scatter(j, b):
            t, l = _slot(j)
            pltpu.make_async_copy(rows_v.at[b], agg_sh.at[dst_v.at[t, l]],
                                  ssem[b]).wait()

        def _start_gather(j, b):
            t, l = _slot(j)
            pltpu.async_copy(table_hbm.at[src_v.at[t, l]], rows_v.at[b],
                             gsem[b])

        def _run(n, base):
            nsect = n // SECT

            def _step_tail(j):
                if j >= 1:
                    _wait_scatter(j - 1, (j - 1) % NBUF)
                _start_gather(j + 2, (j + 2) % NBUF)

            for j in range(3):
                _wait_gather(j, j % NBUF)
                _start_scatter(j, j % NBUF)
                if j == 1 and nsect > 1:
                    for cp in _idx_copies(base + SECT, 1):
                        cp.start()
                _step_tail(j)

            m = 3 * ((n - 2) // 3)

            @pl.loop(3, m, step=3)
            def _(j0):
                for u in range(3):
                    j = j0 + u
                    b = u % NBUF
                    t, l = _slot(j)
                    _wait_gather(j, b)
                    _start_scatter(j, b)
                    _wait_scatter(j - 1, (u - 1) % NBUF)
                    sec = j // SECT

                    @pl.when(jnp.logical_and(l == 1, sec + 1 < nsect))
                    def _():
                        for cp in _idx_copies(base + (sec + 1) * SECT, 1 - t):
                            cp.start()

                    @pl.when(jnp.logical_and(l == SECT - 2, sec + 1 < nsect))
                    def _():
                        for cp in _idx_copies(base + (sec + 1) * SECT, 1 - t):
                            cp.wait()

                    _start_gather(j + 2, (u + 2) % NBUF)

            for j in range(m, n):
                _wait_gather(j, j % NBUF)
                _start_scatter(j, j % NBUF)
                if j + 2 < n:
                    _wait_scatter(j - 1, (j - 1) % NBUF)
                    _start_gather(j + 2, (j + 2) % NBUF)

            for j in range(n - NBUF, n):
                _wait_scatter(j, j % NBUF)

        base0 = s * CH0
        base1 = NC0_CHUNKS + s * CH1
        base_c = jnp.where(c == 0, base0, base1)
        for cp in _idx_copies(base_c, 0):
            cp.start()

        @pl.loop(0, CHUNK)
        def _(i):
            for g in range(F // 16):
                rows_v[0, i, pl.ds(g * 16, 16)] = jnp.zeros((16,), jnp.float32)
        zcp = [
            pltpu.async_copy(rows_v.at[0],
                             agg_sh.at[pl.ds(row0 + t * CHUNK, CHUNK)], bsem)
            for t in range(nseg)
        ]
        for cp in zcp:
            cp.wait()
        for cp in _idx_copies(base_c, 0):
            cp.wait()
        _start_gather(0, 0)
        _start_gather(1, 1)
        plsc.subcore_barrier()

        pl.when(c == 0)(lambda: _run(CH0, base0))
        pl.when(c != 0)(lambda: _run(CH1, base1))
        plsc.subcore_barrier()

        wcp = []
        for t in range(nseg):
            sl = pl.ds(row0 + t * CHUNK, CHUNK)
            wcp.append(pltpu.async_copy(agg_sh.at[sl], out_hbm.at[c, sl], bsem))
        for cp in wcp:
            cp.wait()

    return k(table, src2d, dst2d)


_BM = 1024


def _tc_relu(x):
    def body(x_ref, o_ref):
        o_ref[...] = jnp.maximum(x_ref[...], 0.0)

    return pl.pallas_call(
        body,
        grid=(NP // _BM,),
        in_specs=[pl.BlockSpec((_BM, F), lambda i: (i, 0))],
        out_specs=pl.BlockSpec((_BM, F), lambda i: (i, 0)),
        out_shape=jax.ShapeDtypeStruct((NP, F), jnp.float32),
    )(x)


def _tc_update(x, parts, W, b, final):
    K, M = W.shape

    def body(x_ref, p_ref, w_ref, b_ref, o_ref):
        acc = x_ref[...] + p_ref[0] + p_ref[1]
        z = jax.lax.dot_general(
            acc, w_ref[...], (((1,), (0,)), ((), ())),
            precision=lax.Precision.HIGHEST,
            preferred_element_type=jnp.float32,
        ) + b_ref[...]
        if final:
            m = jnp.max(z, axis=1, keepdims=True)
            e = jnp.exp(z - m)
            o_ref[...] = (z - m) - jnp.log(jnp.sum(e, axis=1, keepdims=True))
        else:
            o_ref[...] = jnp.maximum(z, 0.0)

    return pl.pallas_call(
        body,
        grid=(NP // _BM,),
        in_specs=[
            pl.BlockSpec((_BM, K), lambda i: (i, 0)),
            pl.BlockSpec((NC, _BM, K), lambda i: (0, i, 0)),
            pl.BlockSpec((K, M), lambda i: (0, 0)),
            pl.BlockSpec((1, M), lambda i: (0, 0)),
        ],
        out_specs=pl.BlockSpec((_BM, M), lambda i: (i, 0)),
        out_shape=jax.ShapeDtypeStruct((NP, M), jnp.float32),
    )(x, parts, W, b)


def kernel(x, edge_index, W1, b1, W2, b2):
    x_p = jnp.pad(x, ((0, NP - N), (0, 0)))
    pad = jnp.full((EP - E,), N, jnp.int32)
    src2d = jnp.concatenate([edge_index[0], pad]).reshape(CHUNKS_TOTAL, CHUNK)
    dst2d = jnp.concatenate([edge_index[1], pad]).reshape(CHUNKS_TOTAL, CHUNK)

    relu_x = _tc_relu(x_p)
    parts1 = _sc_scatter(relu_x, src2d, dst2d)
    h = _tc_update(x_p, parts1, W1, b1.reshape(1, H), final=False)
    parts2 = _sc_scatter(h, src2d, dst2d)
    out = _tc_update(h, parts2, W2, b2.reshape(1, C), final=True)
    return out[:N]


# speedup vs baseline: 1.0375x; 1.0375x over previous
import functools

import jax
import jax.numpy as jnp
from jax import lax
from jax.experimental import pallas as pl
from jax.experimental.pallas import tpu as pltpu
from jax.experimental.pallas import tpu_sc as plsc

N, E, F, H, C = 10000, 320000, 128, 128, 64
NP = 10240
CHUNK = 64
EP = 327680
NC, NS = 2, 16
CHUNKS_TOTAL = EP // CHUNK
CHUNKS_PER_CORE = CHUNKS_TOTAL // NC
CHUNKS_PER_TILE = CHUNKS_PER_CORE // NS
ROWS_PER_TILE = NP // NS
SECT = 32
SECT0, SECT1 = 8, 2
NC0_CHUNKS = NS * SECT0 * SECT


def _sc_scatter(table, src2d, dst2d):
    mesh = plsc.VectorSubcoreMesh(core_axis_name="c", subcore_axis_name="s")

    @functools.partial(
        pl.kernel,
        out_type=jax.ShapeDtypeStruct((NC, NP, F), jnp.float32),
        mesh=mesh,
        scratch_types=[
            pltpu.VMEM_SHARED((NP, F), jnp.float32),
            pltpu.VMEM((2, SECT, CHUNK), jnp.int32),
            pltpu.VMEM((2, SECT, CHUNK), jnp.int32),
            pltpu.VMEM((2, CHUNK, F), jnp.float32),
        ] + [pltpu.SemaphoreType.DMA] * 7,
    )
    def k(table_hbm, src_hbm, dst_hbm, out_hbm,
          agg_sh, src_v, dst_v, rows_v, g0, g1, s0, s1, i0, i1, bsem):
        gsem = (g0, g1)
        ssem = (s0, s1)
        isem = (i0, i1)
        c = lax.axis_index("c")
        s = lax.axis_index("s")
        row0 = s * ROWS_PER_TILE
        nseg = ROWS_PER_TILE // CHUNK

        def _idx_copies(sect_chunk0, t):
            return (
                pltpu.make_async_copy(src_hbm.at[pl.ds(sect_chunk0, SECT)],
                                      src_v.at[t], isem[t]),
                pltpu.make_async_copy(dst_hbm.at[pl.ds(sect_chunk0, SECT)],
                                      dst_v.at[t], isem[t]),
            )

        def _wait_gather(t, l, b):
            pltpu.make_async_copy(table_hbm.at[src_v.at[t, l]],
                                  rows_v.at[b], gsem[b]).wait()

        def _start_scatter(t, l, b):
            pltpu.async_copy(rows_v.at[b], agg_sh.at[dst_v.at[t, l]],
                             ssem[b], add=True)

        def _wait_scatter(t, l, b):
            pltpu.make_async_copy(rows_v.at[b], agg_sh.at[dst_v.at[t, l]],
                                  ssem[b]).wait()

        def _start_gather(t, l, b):
            pltpu.async_copy(table_hbm.at[src_v.at[t, l]], rows_v.at[b],
                             gsem[b])

        def _run(nsect, base):
            for sect in range(nsect):
                t = sect % 2
                if sect > 0:
                    for cp in _idx_copies(base + sect * SECT, t):
                        cp.wait()
                if sect + 1 < nsect:
                    for cp in _idx_copies(base + (sect + 1) * SECT, 1 - t):
                        cp.start()

                _start_gather(t, 0, 0)
                _wait_gather(t, 0, 0)
                _start_scatter(t, 0, 0)
                _start_gather(t, 1, 1)

                @pl.loop(1, SECT - 1, step=2)
                def _(j0):
                    for u in range(2):
                        j = j0 + u
                        b = (1 + u) % 2
                        _wait_gather(t, j, b)
                        _start_scatter(t, j, b)
                        _wait_scatter(t, j - 1, 1 - b)
                        _start_gather(t, j + 1, 1 - b)

                jl = SECT - 1
                _wait_gather(t, jl, 1)
                _start_scatter(t, jl, 1)
                _wait_scatter(t, jl - 1, 0)
                _wait_scatter(t, jl, 1)

        base0 = s * SECT0 * SECT
        base1 = NC0_CHUNKS + s * SECT1 * SECT
        base_c = jnp.where(c == 0, base0, base1)
        for cp in _idx_copies(base_c, 0):
            cp.start()

        @pl.loop(0, CHUNK)
        def _(i):
            for g in range(F // 16):
                rows_v[0, i, pl.ds(g * 16, 16)] = jnp.zeros((16,), jnp.float32)
        zcp = [
            pltpu.async_copy(rows_v.at[0],
                             agg_sh.at[pl.ds(row0 + t * CHUNK, CHUNK)], bsem)
            for t in range(nseg)
        ]
        for cp in zcp:
            cp.wait()
        for cp in _idx_copies(base_c, 0):
            cp.wait()
        plsc.subcore_barrier()

        pl.when(c == 0)(lambda: _run(SECT0, base0))
        pl.when(c != 0)(lambda: _run(SECT1, base1))
        plsc.subcore_barrier()

        wcp = []
        for t in range(nseg):
            sl = pl.ds(row0 + t * CHUNK, CHUNK)
            wcp.append(pltpu.async_copy(agg_sh.at[sl], out_hbm.at[c, sl], bsem))
        for cp in wcp:
            cp.wait()

    return k(table, src2d, dst2d)


_BM = 1024


def _tc_relu(x):
    def body(x_ref, o_ref):
        o_ref[...] = jnp.maximum(x_ref[...], 0.0)

    return pl.pallas_call(
        body,
        grid=(NP // _BM,),
        in_specs=[pl.BlockSpec((_BM, F), lambda i: (i, 0))],
        out_specs=pl.BlockSpec((_BM, F), lambda i: (i, 0)),
        out_shape=jax.ShapeDtypeStruct((NP, F), jnp.float32),
    )(x)


def _tc_update(x, parts, W, b, final):
    K, M = W.shape

    def body(x_ref, p_ref, w_ref, b_ref, o_ref):
        acc = x_ref[...] + p_ref[0] + p_ref[1]
        z = jax.lax.dot_general(
            acc, w_ref[...], (((1,), (0,)), ((), ())),
            precision=lax.Precision.HIGHEST,
            preferred_element_type=jnp.float32,
        ) + b_ref[...]
        if final:
            m = jnp.max(z, axis=1, keepdims=True)
            e = jnp.exp(z - m)
            o_ref[...] = (z - m) - jnp.log(jnp.sum(e, axis=1, keepdims=True))
        else:
            o_ref[...] = jnp.maximum(z, 0.0)

    return pl.pallas_call(
        body,
        grid=(NP // _BM,),
        in_specs=[
            pl.BlockSpec((_BM, K), lambda i: (i, 0)),
            pl.BlockSpec((NC, _BM, K), lambda i: (0, i, 0)),
            pl.BlockSpec((K, M), lambda i: (0, 0)),
            pl.BlockSpec((1, M), lambda i: (0, 0)),
        ],
        out_specs=pl.BlockSpec((_BM, M), lambda i: (i, 0)),
        out_shape=jax.ShapeDtypeStruct((NP, M), jnp.float32),
    )(x, parts, W, b)


def kernel(x, edge_index, W1, b1, W2, b2):
    x_p = jnp.pad(x, ((0, NP - N), (0, 0)))
    pad = jnp.full((EP - E,), N, jnp.int32)
    src2d = jnp.concatenate([edge_index[0], pad]).reshape(CHUNKS_TOTAL, CHUNK)
    dst2d = jnp.concatenate([edge_index[1], pad]).reshape(CHUNKS_TOTAL, CHUNK)

    relu_x = _tc_relu(x_p)
    parts1 = _sc_scatter(relu_x, src2d, dst2d)
    h = _tc_update(x_p, parts1, W1, b1.reshape(1, H), final=False)
    parts2 = _sc_scatter(h, src2d, dst2d)
    out = _tc_update(h, parts2, W2, b2.reshape(1, C), final=True)
    return out[:N]


# speedup vs baseline: 1.1650x; 1.1229x over previous
import dataclasses
import functools

import jax
import jax.numpy as jnp
from jax import lax
from jax.experimental import pallas as pl
from jax.experimental.pallas import tpu as pltpu
from jax.experimental.pallas import tpu_sc as plsc

N, E, F, H, C = 10000, 320000, 128, 128, 64
NP = 10240
CHUNK = 64
EP = 327680
NC, NS = 2, 16
CHUNKS_TOTAL = EP // CHUNK
ROWS_PER_TILE = NP // NS
SECT = 32
CH0, CH1 = 256, 64
NC0_CHUNKS = NS * CH0
FH = F // 2
HC = CHUNK // 2


def _sc_scatter(table, src2d, dst2x):
    mesh = plsc.VectorSubcoreMesh(core_axis_name="c", subcore_axis_name="s")
    cp = pltpu.CompilerParams()
    for fld, val in (("needs_layout_passes", False),
                     ("use_tc_tiling_on_sc", False)):
        if fld in pltpu.CompilerParams.__dataclass_fields__:
            cp = dataclasses.replace(cp, **{fld: val})

    @functools.partial(
        pl.kernel,
        out_type=jax.ShapeDtypeStruct((NC, NP, F), jnp.float32),
        mesh=mesh,
        compiler_params=cp,
        scratch_types=[
            pltpu.VMEM_SHARED((NP, F), jnp.float32),
            pltpu.VMEM((2, SECT, CHUNK), jnp.int32),
            pltpu.VMEM((2, 2 * SECT, HC), jnp.int32),
            pltpu.VMEM((2, CHUNK, FH), jnp.int32),
            pltpu.VMEM((2, HC, F), jnp.float32),
        ] + [pltpu.SemaphoreType.DMA] * 6,
    )
    def k(table_hbm, src_hbm, dst_hbm, out_hbm,
          agg_sh, src_v, dst_v, ibuf, fbuf, g0, g1, s0, s1, isem, bsem):
        gsem = (g0, g1)
        ssem = (s0, s1)
        c = lax.axis_index("c")
        s = lax.axis_index("s")
        row0 = s * ROWS_PER_TILE

        def _idx_copies(sect0, t):
            return (
                pltpu.make_async_copy(src_hbm.at[pl.ds(sect0, SECT)],
                                      src_v.at[t], isem),
                pltpu.make_async_copy(dst_hbm.at[pl.ds(2 * sect0, 2 * SECT)],
                                      dst_v.at[t], isem),
            )

        def _wait_gather(t, l, b):
            pltpu.make_async_copy(table_hbm.at[src_v.at[t, l]],
                                  ibuf.at[b], gsem[b]).wait()

        def _start_gather(t, l, b):
            pltpu.async_copy(table_hbm.at[src_v.at[t, l]], ibuf.at[b],
                             gsem[b])

        def _start_scatter(t, l, h, fb):
            pltpu.async_copy(fbuf.at[fb], agg_sh.at[dst_v.at[t, 2 * l + h]],
                             ssem[fb], add=True)

        def _wait_scatter(t, l, h, fb):
            pltpu.make_async_copy(fbuf.at[fb],
                                  agg_sh.at[dst_v.at[t, 2 * l + h]],
                                  ssem[fb]).wait()

        def _convert(b, h, fb):
            @pl.loop(0, HC)
            def _(r):
                for g in range(FH // 16):
                    w = ibuf[b, h * HC + r, pl.ds(g * 16, 16)]
                    fbuf[fb, r, pl.ds(g * 16, 16)] = plsc.bitcast(
                        w << 16, jnp.float32)
                    fbuf[fb, r, pl.ds(FH + g * 16, 16)] = plsc.bitcast(
                        w & jnp.int32(-65536), jnp.float32)

        def _step(t, l, b, first, last):
            _wait_gather(t, l, b)
            for h, fb in ((0, 0), (1, 1)):
                if not first:
                    _wait_scatter(t, l - 1, h, fb)
                _convert(b, h, fb)
                _start_scatter(t, l, h, fb)
            if not last:
                _start_gather(t, l + 2, b)

        def _run(nsect, base):
            for sect in range(nsect):
                t = sect % 2
                if sect > 0:
                    for cp in _idx_copies(base + sect * SECT, t):
                        cp.wait()
                if sect + 1 < nsect:
                    for cp in _idx_copies(base + (sect + 1) * SECT, 1 - t):
                        cp.start()
                if sect > 0:
                    _start_gather(t, 0, 0)
                _start_gather(t, 1, 1)

                _step(t, 0, 0, first=True, last=False)
                _step(t, 1, 1, first=False, last=False)

                @pl.loop(2, SECT - 2, step=2)
                def _(l0):
                    for u in range(2):
                        _step(t, l0 + u, u, first=False, last=False)

                _step(t, SECT - 2, 0, first=False, last=True)
                _step(t, SECT - 1, 1, first=False, last=True)
                for h, fb in ((0, 0), (1, 1)):
                    _wait_scatter(t, SECT - 1, h, fb)

        base0 = s * CH0
        base1 = NC0_CHUNKS + s * CH1
        base_c = jnp.where(c == 0, base0, base1)
        for cp in _idx_copies(base_c, 0):
            cp.start()

        @pl.loop(0, HC)
        def _(i):
            for g in range(F // 16):
                fbuf[0, i, pl.ds(g * 16, 16)] = jnp.zeros((16,), jnp.float32)
        zcp = [
            pltpu.async_copy(fbuf.at[0],
                             agg_sh.at[pl.ds(row0 + t * HC, HC)], bsem)
            for t in range(ROWS_PER_TILE // HC)
        ]
        for cp in zcp:
            cp.wait()
        for cp in _idx_copies(base_c, 0):
            cp.wait()
        _start_gather(0, 0, 0)
        plsc.subcore_barrier()

        pl.when(c == 0)(lambda: _run(CH0 // SECT, base0))
        pl.when(c != 0)(lambda: _run(CH1 // SECT, base1))
        plsc.subcore_barrier()

        wcp = []
        for t in range(ROWS_PER_TILE // CHUNK):
            sl = pl.ds(row0 + t * CHUNK, CHUNK)
            wcp.append(pltpu.async_copy(agg_sh.at[sl], out_hbm.at[c, sl], bsem))
        for cp in wcp:
            cp.wait()

    return k(table, src2d, dst2x)


_BM = 1024


def _pack_rows(z):
    lo = jax.lax.bitcast_convert_type(
        z[:, :FH].astype(jnp.bfloat16), jnp.uint16).astype(jnp.uint32)
    hi = jax.lax.bitcast_convert_type(
        z[:, FH:].astype(jnp.bfloat16), jnp.uint16).astype(jnp.uint32)
    return jax.lax.bitcast_convert_type(lo | (hi << 16), jnp.int32)


def _tc_relu_pack(x):
    def body(x_ref, o_ref):
        o_ref[...] = _pack_rows(jnp.maximum(x_ref[...], 0.0))

    return pl.pallas_call(
        body,
        grid=(NP // _BM,),
        in_specs=[pl.BlockSpec((_BM, F), lambda i: (i, 0))],
        out_specs=pl.BlockSpec((_BM, FH), lambda i: (i, 0)),
        out_shape=jax.ShapeDtypeStruct((NP, FH), jnp.int32),
    )(x)


def _tc_update(x, parts, W, b, final):
    K, M = W.shape

    def body(x_ref, p_ref, w_ref, b_ref, *o_refs):
        acc = x_ref[...] + p_ref[0] + p_ref[1]
        z = jax.lax.dot_general(
            acc, w_ref[...], (((1,), (0,)), ((), ())),
            precision=lax.Precision.HIGHEST,
            preferred_element_type=jnp.float32,
        ) + b_ref[...]
        if final:
            m = jnp.max(z, axis=1, keepdims=True)
            e = jnp.exp(z - m)
            o_refs[0][...] = (z - m) - jnp.log(jnp.sum(e, axis=1, keepdims=True))
        else:
            zr = jnp.maximum(z, 0.0)
            o_refs[0][...] = zr
            o_refs[1][...] = _pack_rows(zr)

    if final:
        out_shape = jax.ShapeDtypeStruct((NP, M), jnp.float32)
        out_specs = pl.BlockSpec((_BM, M), lambda i: (i, 0))
    else:
        out_shape = (jax.ShapeDtypeStruct((NP, M), jnp.float32),
                     jax.ShapeDtypeStruct((NP, M // 2), jnp.int32))
        out_specs = (pl.BlockSpec((_BM, M), lambda i: (i, 0)),
                     pl.BlockSpec((_BM, M // 2), lambda i: (i, 0)))

    return pl.pallas_call(
        body,
        grid=(NP // _BM,),
        in_specs=[
            pl.BlockSpec((_BM, K), lambda i: (i, 0)),
            pl.BlockSpec((NC, _BM, K), lambda i: (0, i, 0)),
            pl.BlockSpec((K, M), lambda i: (0, 0)),
            pl.BlockSpec((1, M), lambda i: (0, 0)),
        ],
        out_specs=out_specs,
        out_shape=out_shape,
    )(x, parts, W, b)


def kernel(x, edge_index, W1, b1, W2, b2):
    x_p = jnp.pad(x, ((0, NP - N), (0, 0)))
    pad = jnp.full((EP - E,), N, jnp.int32)
    src2d = jnp.concatenate([edge_index[0], pad]).reshape(CHUNKS_TOTAL, CHUNK)
    dst2x = jnp.concatenate([edge_index[1], pad]).reshape(2 * CHUNKS_TOTAL, HC)

    packed_x = _tc_relu_pack(x_p)
    parts1 = _sc_scatter(packed_x, src2d, dst2x)
    h, packed_h = _tc_update(x_p, parts1, W1, b1.reshape(1, H), final=False)
    parts2 = _sc_scatter(packed_h, src2d, dst2x)
    out = _tc_update(h, parts2, W2, b2.reshape(1, C), final=True)
    return out[:N]


# speedup vs baseline: 1.3112x; 1.1255x over previous
import dataclasses
import functools

import jax
import jax.numpy as jnp
from jax import lax
from jax.experimental import pallas as pl
from jax.experimental.pallas import tpu as pltpu
from jax.experimental.pallas import tpu_sc as plsc

N, E, F, H, C = 10000, 320000, 128, 128, 64
NP = 10240
CHUNK = 64
EP = 327680
NC, NS = 2, 16
CHUNKS_TOTAL = EP // CHUNK
ROWS_PER_TILE = NP // NS
SECT = 32
CH0, CH1 = 224, 96
NC0_CHUNKS = NS * CH0
FH = F // 2
HC = CHUNK // 2


def _sc_scatter(table, src2d, dst2x):
    mesh = plsc.VectorSubcoreMesh(core_axis_name="c", subcore_axis_name="s")
    cp = pltpu.CompilerParams()
    for fld, val in (("needs_layout_passes", False),
                     ("use_tc_tiling_on_sc", False)):
        if fld in pltpu.CompilerParams.__dataclass_fields__:
            cp = dataclasses.replace(cp, **{fld: val})

    @functools.partial(
        pl.kernel,
        out_type=jax.ShapeDtypeStruct((NC, NP, F), jnp.float32),
        mesh=mesh,
        compiler_params=cp,
        scratch_types=[
            pltpu.VMEM_SHARED((NP, F), jnp.float32),
            pltpu.VMEM((2, SECT, CHUNK), jnp.int32),
            pltpu.VMEM((2, 2 * SECT, HC), jnp.int32),
            pltpu.VMEM((2, CHUNK, FH), jnp.int32),
            pltpu.VMEM((2, HC, F), jnp.float32),
        ] + [pltpu.SemaphoreType.DMA] * 6,
    )
    def k(table_hbm, src_hbm, dst_hbm, out_hbm,
          agg_sh, src_v, dst_v, ibuf, fbuf, g0, g1, s0, s1, isem, bsem):
        gsem = (g0, g1)
        ssem = (s0, s1)
        c = lax.axis_index("c")
        s = lax.axis_index("s")
        row0 = s * ROWS_PER_TILE

        def _idx_copies(sect0, t):
            return (
                pltpu.make_async_copy(src_hbm.at[pl.ds(sect0, SECT)],
                                      src_v.at[t], isem),
                pltpu.make_async_copy(dst_hbm.at[pl.ds(2 * sect0, 2 * SECT)],
                                      dst_v.at[t], isem),
            )

        def _wait_gather(t, l, b):
            pltpu.make_async_copy(table_hbm.at[src_v.at[t, l]],
                                  ibuf.at[b], gsem[b]).wait()

        def _start_gather(t, l, b):
            pltpu.async_copy(table_hbm.at[src_v.at[t, l]], ibuf.at[b],
                             gsem[b])

        def _start_scatter(t, l, h, fb):
            pltpu.async_copy(fbuf.at[fb], agg_sh.at[dst_v.at[t, 2 * l + h]],
                             ssem[fb], add=True)

        def _wait_scatter(t, l, h, fb):
            pltpu.make_async_copy(fbuf.at[fb],
                                  agg_sh.at[dst_v.at[t, 2 * l + h]],
                                  ssem[fb]).wait()

        def _convert(b, h, fb):
            @pl.loop(0, HC)
            def _(r):
                for g in range(FH // 16):
                    w = ibuf[b, h * HC + r, pl.ds(g * 16, 16)]
                    fbuf[fb, r, pl.ds(g * 16, 16)] = plsc.bitcast(
                        w << 16, jnp.float32)
                    fbuf[fb, r, pl.ds(FH + g * 16, 16)] = plsc.bitcast(
                        w & jnp.int32(-65536), jnp.float32)

        def _step(t, l, b, first, last):
            _wait_gather(t, l, b)
            for h, fb in ((0, 0), (1, 1)):
                if not first:
                    _wait_scatter(t, l - 1, h, fb)
                _convert(b, h, fb)
                _start_scatter(t, l, h, fb)
            if not last:
                _start_gather(t, l + 2, b)

        def _run(nsect, base):
            for sect in range(nsect):
                t = sect % 2
                if sect > 0:
                    for cp in _idx_copies(base + sect * SECT, t):
                        cp.wait()
                if sect + 1 < nsect:
                    for cp in _idx_copies(base + (sect + 1) * SECT, 1 - t):
                        cp.start()
                if sect > 0:
                    _start_gather(t, 0, 0)
                _start_gather(t, 1, 1)

                _step(t, 0, 0, first=True, last=False)
                _step(t, 1, 1, first=False, last=False)

                @pl.loop(2, SECT - 2, step=2)
                def _(l0):
                    for u in range(2):
                        _step(t, l0 + u, u, first=False, last=False)

                _step(t, SECT - 2, 0, first=False, last=True)
                _step(t, SECT - 1, 1, first=False, last=True)
                for h, fb in ((0, 0), (1, 1)):
                    _wait_scatter(t, SECT - 1, h, fb)

        base0 = s * CH0
        base1 = NC0_CHUNKS + s * CH1
        base_c = jnp.where(c == 0, base0, base1)
        for cp in _idx_copies(base_c, 0):
            cp.start()

        @pl.loop(0, HC)
        def _(i):
            for g in range(F // 16):
                fbuf[0, i, pl.ds(g * 16, 16)] = jnp.zeros((16,), jnp.float32)
        zcp = [
            pltpu.async_copy(fbuf.at[0],
                             agg_sh.at[pl.ds(row0 + t * HC, HC)], bsem)
            for t in range(ROWS_PER_TILE // HC)
        ]
        for cp in zcp:
            cp.wait()
        for cp in _idx_copies(base_c, 0):
            cp.wait()
        _start_gather(0, 0, 0)
        plsc.subcore_barrier()

        pl.when(c == 0)(lambda: _run(CH0 // SECT, base0))
        pl.when(c != 0)(lambda: _run(CH1 // SECT, base1))
        plsc.subcore_barrier()

        wcp = []
        for t in range(ROWS_PER_TILE // CHUNK):
            sl = pl.ds(row0 + t * CHUNK, CHUNK)
            wcp.append(pltpu.async_copy(agg_sh.at[sl], out_hbm.at[c, sl], bsem))
        for cp in wcp:
            cp.wait()

    return k(table, src2d, dst2x)


_BM = 1024


def _pack_rows(z):
    lo = jax.lax.bitcast_convert_type(
        z[:, :FH].astype(jnp.bfloat16), jnp.uint16).astype(jnp.uint32)
    hi = jax.lax.bitcast_convert_type(
        z[:, FH:].astype(jnp.bfloat16), jnp.uint16).astype(jnp.uint32)
    return jax.lax.bitcast_convert_type(lo | (hi << 16), jnp.int32)


def _tc_relu_pack(x):
    def body(x_ref, o_ref):
        o_ref[...] = _pack_rows(jnp.maximum(x_ref[...], 0.0))

    return pl.pallas_call(
        body,
        grid=(NP // _BM,),
        in_specs=[pl.BlockSpec((_BM, F), lambda i: (i, 0))],
        out_specs=pl.BlockSpec((_BM, FH), lambda i: (i, 0)),
        out_shape=jax.ShapeDtypeStruct((NP, FH), jnp.int32),
    )(x)


def _tc_update(x, parts, W, b, final):
    K, M = W.shape

    def body(x_ref, p_ref, w_ref, b_ref, *o_refs):
        acc = x_ref[...] + p_ref[0] + p_ref[1]
        z = jax.lax.dot_general(
            acc, w_ref[...], (((1,), (0,)), ((), ())),
            precision=lax.Precision.HIGHEST,
            preferred_element_type=jnp.float32,
        ) + b_ref[...]
        if final:
            m = jnp.max(z, axis=1, keepdims=True)
            e = jnp.exp(z - m)
            o_refs[0][...] = (z - m) - jnp.log(jnp.sum(e, axis=1, keepdims=True))
        else:
            zr = jnp.maximum(z, 0.0)
            o_refs[0][...] = zr
            o_refs[1][...] = _pack_rows(zr)

    if final:
        out_shape = jax.ShapeDtypeStruct((NP, M), jnp.float32)
        out_specs = pl.BlockSpec((_BM, M), lambda i: (i, 0))
    else:
        out_shape = (jax.ShapeDtypeStruct((NP, M), jnp.float32),
                     jax.ShapeDtypeStruct((NP, M // 2), jnp.int32))
        out_specs = (pl.BlockSpec((_BM, M), lambda i: (i, 0)),
                     pl.BlockSpec((_BM, M // 2), lambda i: (i, 0)))

    return pl.pallas_call(
        body,
        grid=(NP // _BM,),
        in_specs=[
            pl.BlockSpec((_BM, K), lambda i: (i, 0)),
            pl.BlockSpec((NC, _BM, K), lambda i: (0, i, 0)),
            pl.BlockSpec((K, M), lambda i: (0, 0)),
            pl.BlockSpec((1, M), lambda i: (0, 0)),
        ],
        out_specs=out_specs,
        out_shape=out_shape,
    )(x, parts, W, b)


def kernel(x, edge_index, W1, b1, W2, b2):
    x_p = jnp.pad(x, ((0, NP - N), (0, 0)))
    pad = jnp.full((EP - E,), N, jnp.int32)
    src2d = jnp.concatenate([edge_index[0], pad]).reshape(CHUNKS_TOTAL, CHUNK)
    dst2x = jnp.concatenate([edge_index[1], pad]).reshape(2 * CHUNKS_TOTAL, HC)

    packed_x = _tc_relu_pack(x_p)
    parts1 = _sc_scatter(packed_x, src2d, dst2x)
    h, packed_h = _tc_update(x_p, parts1, W1, b1.reshape(1, H), final=False)
    parts2 = _sc_scatter(packed_h, src2d, dst2x)
    out = _tc_update(h, parts2, W2, b2.reshape(1, C), final=True)
    return out[:N]


# speedup vs baseline: 1.3249x; 1.0104x over previous
import dataclasses
import functools

import jax
import jax.numpy as jnp
from jax import lax
from jax.experimental import pallas as pl
from jax.experimental.pallas import tpu as pltpu
from jax.experimental.pallas import tpu_sc as plsc

N, E, F, H, C = 10000, 320000, 128, 128, 64
NP = 10240
CHUNK = 64
EP = 327680
NC, NS = 2, 16
CHUNKS_TOTAL = EP // CHUNK
ROWS_PER_TILE = NP // NS
SECT = 32
CH0, CH1 = 224, 96
NC0_CHUNKS = NS * CH0
FH = F // 2
HC = CHUNK // 2


def _sc_scatter(table, src2d, dst2x):
    mesh = plsc.VectorSubcoreMesh(core_axis_name="c", subcore_axis_name="s")
    cp = pltpu.CompilerParams()
    for fld, val in (("needs_layout_passes", False),
                     ("use_tc_tiling_on_sc", False)):
        if fld in pltpu.CompilerParams.__dataclass_fields__:
            cp = dataclasses.replace(cp, **{fld: val})

    @functools.partial(
        pl.kernel,
        out_type=jax.ShapeDtypeStruct((NC, NP, F), jnp.float32),
        mesh=mesh,
        compiler_params=cp,
        scratch_types=[
            pltpu.VMEM_SHARED((NP, F), jnp.float32),
            pltpu.VMEM((2, SECT, CHUNK), jnp.int32),
            pltpu.VMEM((2, 2 * SECT, HC), jnp.int32),
            pltpu.VMEM((2, CHUNK, FH), jnp.int32),
            pltpu.VMEM((2, HC, F), jnp.float32),
        ] + [pltpu.SemaphoreType.DMA] * 6,
    )
    def k(table_hbm, src_hbm, dst_hbm, out_hbm,
          agg_sh, src_v, dst_v, ibuf, fbuf, g0, g1, s0, s1, isem, bsem):
        gsem = (g0, g1)
        ssem = (s0, s1)
        c = lax.axis_index("c")
        s = lax.axis_index("s")
        row0 = s * ROWS_PER_TILE

        def _idx_copies(sect0, t):
            return (
                pltpu.make_async_copy(src_hbm.at[pl.ds(sect0, SECT)],
                                      src_v.at[t], isem),
                pltpu.make_async_copy(dst_hbm.at[pl.ds(2 * sect0, 2 * SECT)],
                                      dst_v.at[t], isem),
            )

        def _wait_gather(t, l, b):
            pltpu.make_async_copy(table_hbm.at[src_v.at[t, l]],
                                  ibuf.at[b], gsem[b]).wait()

        def _start_gather(t, l, b):
            pltpu.async_copy(table_hbm.at[src_v.at[t, l]], ibuf.at[b],
                             gsem[b])

        def _start_scatter(t, l, h, fb):
            pltpu.async_copy(fbuf.at[fb], agg_sh.at[dst_v.at[t, 2 * l + h]],
                             ssem[fb], add=True)

        def _wait_scatter(t, l, h, fb):
            pltpu.make_async_copy(fbuf.at[fb],
                                  agg_sh.at[dst_v.at[t, 2 * l + h]],
                                  ssem[fb]).wait()

        def _convert(b, h, fb):
            @pl.loop(0, HC, step=4)
            def _(r0):
                for dr in range(4):
                    r = r0 + dr
                    for g in range(FH // 16):
                        w = ibuf[b, h * HC + r, pl.ds(g * 16, 16)]
                        fbuf[fb, r, pl.ds(g * 16, 16)] = plsc.bitcast(
                            w << 16, jnp.float32)
                        fbuf[fb, r, pl.ds(FH + g * 16, 16)] = plsc.bitcast(
                            w & jnp.int32(-65536), jnp.float32)

        def _step(t, l, b, first, last):
            _wait_gather(t, l, b)
            for h, fb in ((0, 0), (1, 1)):
                if not first:
                    _wait_scatter(t, l - 1, h, fb)
                _convert(b, h, fb)
                _start_scatter(t, l, h, fb)
            if not last:
                _start_gather(t, l + 2, b)

        def _run(nsect, base):
            @pl.loop(0, nsect)
            def _(sect):
                t = sect % 2

                def _retire_prefetch():
                    for cp in _idx_copies(base + sect * SECT, t):
                        cp.wait()

                def _prefetch_next():
                    for cp in _idx_copies(base + (sect + 1) * SECT, 1 - t):
                        cp.start()

                pl.when(sect > 0)(_retire_prefetch)
                pl.when(sect + 1 < nsect)(_prefetch_next)
                pl.when(sect > 0)(lambda: _start_gather(t, 0, 0))
                _start_gather(t, 1, 1)

                _step(t, 0, 0, first=True, last=False)
                _step(t, 1, 1, first=False, last=False)

                @pl.loop(2, SECT - 2, step=2)
                def _(l0):
                    for u in range(2):
                        _step(t, l0 + u, u, first=False, last=False)

                _step(t, SECT - 2, 0, first=False, last=True)
                _step(t, SECT - 1, 1, first=False, last=True)
                for h, fb in ((0, 0), (1, 1)):
                    _wait_scatter(t, SECT - 1, h, fb)

        base0 = s * CH0
        base1 = NC0_CHUNKS + s * CH1
        base_c = jnp.where(c == 0, base0, base1)
        for cp in _idx_copies(base_c, 0):
            cp.start()

        @pl.loop(0, HC)
        def _(i):
            for g in range(F // 16):
                fbuf[0, i, pl.ds(g * 16, 16)] = jnp.zeros((16,), jnp.float32)
        zcp = [
            pltpu.async_copy(fbuf.at[0],
                             agg_sh.at[pl.ds(row0 + t * HC, HC)], bsem)
            for t in range(ROWS_PER_TILE // HC)
        ]
        for cp in zcp:
            cp.wait()
        for cp in _idx_copies(base_c, 0):
            cp.wait()
        _start_gather(0, 0, 0)
        plsc.subcore_barrier()

        nsect_c = jnp.where(c == 0, CH0 // SECT, CH1 // SECT)
        _run(nsect_c, base_c)
        plsc.subcore_barrier()

        wcp = []
        for t in range(ROWS_PER_TILE // CHUNK):
            sl = pl.ds(row0 + t * CHUNK, CHUNK)
            wcp.append(pltpu.async_copy(agg_sh.at[sl], out_hbm.at[c, sl], bsem))
        for cp in wcp:
            cp.wait()

    return k(table, src2d, dst2x)


_BM = 1024


def _pack_rows(z):
    lo = jax.lax.bitcast_convert_type(
        z[:, :FH].astype(jnp.bfloat16), jnp.uint16).astype(jnp.uint32)
    hi = jax.lax.bitcast_convert_type(
        z[:, FH:].astype(jnp.bfloat16), jnp.uint16).astype(jnp.uint32)
    return jax.lax.bitcast_convert_type(lo | (hi << 16), jnp.int32)


def _tc_relu_pack(x):
    def body(x_ref, o_ref):
        o_ref[...] = _pack_rows(jnp.maximum(x_ref[...], 0.0))

    return pl.pallas_call(
        body,
        grid=(NP // _BM,),
        in_specs=[pl.BlockSpec((_BM, F), lambda i: (i, 0))],
        out_specs=pl.BlockSpec((_BM, FH), lambda i: (i, 0)),
        out_shape=jax.ShapeDtypeStruct((NP, FH), jnp.int32),
    )(x)


def _tc_update(x, parts, W, b, final):
    K, M = W.shape

    def body(x_ref, p_ref, w_ref, b_ref, *o_refs):
        acc = x_ref[...] + p_ref[0] + p_ref[1]
        z = jax.lax.dot_general(
            acc, w_ref[...], (((1,), (0,)), ((), ())),
            precision=lax.Precision.HIGHEST,
            preferred_element_type=jnp.float32,
        ) + b_ref[...]
        if final:
            m = jnp.max(z, axis=1, keepdims=True)
            e = jnp.exp(z - m)
            o_refs[0][...] = (z - m) - jnp.log(jnp.sum(e, axis=1, keepdims=True))
        else:
            zr = jnp.maximum(z, 0.0)
            o_refs[0][...] = zr
            o_refs[1][...] = _pack_rows(zr)

    if final:
        out_shape = jax.ShapeDtypeStruct((NP, M), jnp.float32)
        out_specs = pl.BlockSpec((_BM, M), lambda i: (i, 0))
    else:
        out_shape = (jax.ShapeDtypeStruct((NP, M), jnp.float32),
                     jax.ShapeDtypeStruct((NP, M // 2), jnp.int32))
        out_specs = (pl.BlockSpec((_BM, M), lambda i: (i, 0)),
                     pl.BlockSpec((_BM, M // 2), lambda i: (i, 0)))

    return pl.pallas_call(
        body,
        grid=(NP // _BM,),
        in_specs=[
            pl.BlockSpec((_BM, K), lambda i: (i, 0)),
            pl.BlockSpec((NC, _BM, K), lambda i: (0, i, 0)),
            pl.BlockSpec((K, M), lambda i: (0, 0)),
            pl.BlockSpec((1, M), lambda i: (0, 0)),
        ],
        out_specs=out_specs,
        out_shape=out_shape,
    )(x, parts, W, b)


def kernel(x, edge_index, W1, b1, W2, b2):
    x_p = jnp.pad(x, ((0, NP - N), (0, 0)))
    pad = jnp.full((EP - E,), N, jnp.int32)
    src2d = jnp.concatenate([edge_index[0], pad]).reshape(CHUNKS_TOTAL, CHUNK)
    dst2x = jnp.concatenate([edge_index[1], pad]).reshape(2 * CHUNKS_TOTAL, HC)

    packed_x = _tc_relu_pack(x_p)
    parts1 = _sc_scatter(packed_x, src2d, dst2x)
    h, packed_h = _tc_update(x_p, parts1, W1, b1.reshape(1, H), final=False)
    parts2 = _sc_scatter(packed_h, src2d, dst2x)
    out = _tc_update(h, parts2, W2, b2.reshape(1, C), final=True)
    return out[:N]


# speedup vs baseline: 1.5143x; 1.1429x over previous
import dataclasses
import functools

import jax
import jax.numpy as jnp
from jax import lax
from jax.experimental import pallas as pl
from jax.experimental.pallas import tpu as pltpu
from jax.experimental.pallas import tpu_sc as plsc

N, E, F, H, C = 10000, 320000, 128, 128, 64
NP = 10240
CHUNK = 64
EP = 327680
NC, NS = 2, 16
CHUNKS_TOTAL = EP // CHUNK
ROWS_PER_TILE = NP // NS
SECT = 32
CH0, CH1 = 192, 128
NC0_CHUNKS = NS * CH0
FH = F // 2
HC = CHUNK // 2


def _sc_scatter(table, src2d, dst2x):
    mesh = plsc.VectorSubcoreMesh(core_axis_name="c", subcore_axis_name="s")
    cp = pltpu.CompilerParams()
    for fld, val in (("needs_layout_passes", False),
                     ("use_tc_tiling_on_sc", False)):
        if fld in pltpu.CompilerParams.__dataclass_fields__:
            cp = dataclasses.replace(cp, **{fld: val})

    @functools.partial(
        pl.kernel,
        out_type=jax.ShapeDtypeStruct((NC, NP, F), jnp.float32),
        mesh=mesh,
        compiler_params=cp,
        scratch_types=[
            pltpu.VMEM_SHARED((NP, F), jnp.float32),
            pltpu.VMEM((2, SECT, CHUNK), jnp.int32),
            pltpu.VMEM((2, 2 * SECT, HC), jnp.int32),
            pltpu.VMEM((2, CHUNK, FH), jnp.int32),
            pltpu.VMEM((2, HC, F), jnp.float32),
        ] + [pltpu.SemaphoreType.DMA] * 6,
    )
    def k(table_hbm, src_hbm, dst_hbm, out_hbm,
          agg_sh, src_v, dst_v, ibuf, fbuf, g0, g1, s0, s1, isem, bsem):
        gsem = (g0, g1)
        ssem = (s0, s1)
        c = lax.axis_index("c")
        s = lax.axis_index("s")
        row0 = s * ROWS_PER_TILE

        def _idx_copies(sect0, t):
            return (
                pltpu.make_async_copy(src_hbm.at[pl.ds(sect0, SECT)],
                                      src_v.at[t], isem),
                pltpu.make_async_copy(dst_hbm.at[pl.ds(2 * sect0, 2 * SECT)],
                                      dst_v.at[t], isem),
            )

        def _wait_gather(t, l, b):
            pltpu.make_async_copy(table_hbm.at[src_v.at[t, l]],
                                  ibuf.at[b], gsem[b]).wait()

        def _start_gather(t, l, b):
            pltpu.async_copy(table_hbm.at[src_v.at[t, l]], ibuf.at[b],
                             gsem[b])

        def _start_scatter(t, l, h, fb):
            pltpu.async_copy(fbuf.at[fb], agg_sh.at[dst_v.at[t, 2 * l + h]],
                             ssem[fb], add=True)

        def _wait_scatter(t, l, h, fb):
            pltpu.make_async_copy(fbuf.at[fb],
                                  agg_sh.at[dst_v.at[t, 2 * l + h]],
                                  ssem[fb]).wait()

        def _convert(b, h, fb):
            @pl.loop(0, HC, step=4)
            def _(r0):
                for dr in range(4):
                    r = r0 + dr
                    for g in range(FH // 16):
                        w = ibuf[b, h * HC + r, pl.ds(g * 16, 16)]
                        fbuf[fb, r, pl.ds(g * 16, 16)] = plsc.bitcast(
                            w << 16, jnp.float32)
                        fbuf[fb, r, pl.ds(FH + g * 16, 16)] = plsc.bitcast(
                            w & jnp.int32(-65536), jnp.float32)

        def _step(t, l, b, first, last):
            _wait_gather(t, l, b)
            for h, fb in ((0, 0), (1, 1)):
                if not first:
                    _wait_scatter(t, l - 1, h, fb)
                _convert(b, h, fb)
                _start_scatter(t, l, h, fb)
            if not last:
                _start_gather(t, l + 2, b)

        def _run(nsect, base):
            @pl.loop(0, nsect)
            def _(sect):
                t = sect % 2

                def _retire_prefetch():
                    for cp in _idx_copies(base + sect * SECT, t):
                        cp.wait()

                def _prefetch_next():
                    for cp in _idx_copies(base + (sect + 1) * SECT, 1 - t):
                        cp.start()

                pl.when(sect > 0)(_retire_prefetch)
                pl.when(sect + 1 < nsect)(_prefetch_next)
                pl.when(sect > 0)(lambda: _start_gather(t, 0, 0))
                _start_gather(t, 1, 1)

                _step(t, 0, 0, first=True, last=False)
                _step(t, 1, 1, first=False, last=False)

                @pl.loop(2, SECT - 2, step=2)
                def _(l0):
                    for u in range(2):
                        _step(t, l0 + u, u, first=False, last=False)

                _step(t, SECT - 2, 0, first=False, last=True)
                _step(t, SECT - 1, 1, first=False, last=True)
                for h, fb in ((0, 0), (1, 1)):
                    _wait_scatter(t, SECT - 1, h, fb)

        base0 = s * CH0
        base1 = NC0_CHUNKS + s * CH1
        base_c = jnp.where(c == 0, base0, base1)
        for cp in _idx_copies(base_c, 0):
            cp.start()

        @pl.loop(0, HC)
        def _(i):
            for g in range(F // 16):
                fbuf[0, i, pl.ds(g * 16, 16)] = jnp.zeros((16,), jnp.float32)
        zcp = [
            pltpu.async_copy(fbuf.at[0],
                             agg_sh.at[pl.ds(row0 + t * HC, HC)], bsem)
            for t in range(ROWS_PER_TILE // HC)
        ]
        for cp in zcp:
            cp.wait()
        for cp in _idx_copies(base_c, 0):
            cp.wait()
        _start_gather(0, 0, 0)
        plsc.subcore_barrier()

        nsect_c = jnp.where(c == 0, CH0 // SECT, CH1 // SECT)
        _run(nsect_c, base_c)
        plsc.subcore_barrier()

        wcp = []
        for t in range(ROWS_PER_TILE // CHUNK):
            sl = pl.ds(row0 + t * CHUNK, CHUNK)
            wcp.append(pltpu.async_copy(agg_sh.at[sl], out_hbm.at[c, sl], bsem))
        for cp in wcp:
            cp.wait()

    return k(table, src2d, dst2x)


_BM = 1024


def _pack_rows(z):
    lo = jax.lax.bitcast_convert_type(
        z[:, :FH].astype(jnp.bfloat16), jnp.uint16).astype(jnp.uint32)
    hi = jax.lax.bitcast_convert_type(
        z[:, FH:].astype(jnp.bfloat16), jnp.uint16).astype(jnp.uint32)
    return jax.lax.bitcast_convert_type(lo | (hi << 16), jnp.int32)


def _tc_relu_pack(x):
    def body(x_ref, o_ref):
        o_ref[...] = _pack_rows(jnp.maximum(x_ref[...], 0.0))

    return pl.pallas_call(
        body,
        grid=(NP // _BM,),
        in_specs=[pl.BlockSpec((_BM, F), lambda i: (i, 0))],
        out_specs=pl.BlockSpec((_BM, FH), lambda i: (i, 0)),
        out_shape=jax.ShapeDtypeStruct((NP, FH), jnp.int32),
    )(x)


def _tc_update(x, parts, W, b, final):
    K, M = W.shape

    def body(x_ref, p_ref, w_ref, b_ref, *o_refs):
        acc = x_ref[...] + p_ref[0] + p_ref[1]
        z = jax.lax.dot_general(
            acc, w_ref[...], (((1,), (0,)), ((), ())),
            precision=lax.Precision.HIGHEST,
            preferred_element_type=jnp.float32,
        ) + b_ref[...]
        if final:
            m = jnp.max(z, axis=1, keepdims=True)
            e = jnp.exp(z - m)
            o_refs[0][...] = (z - m) - jnp.log(jnp.sum(e, axis=1, keepdims=True))
        else:
            zr = jnp.maximum(z, 0.0)
            o_refs[0][...] = zr
            o_refs[1][...] = _pack_rows(zr)

    if final:
        out_shape = jax.ShapeDtypeStruct((NP, M), jnp.float32)
        out_specs = pl.BlockSpec((_BM, M), lambda i: (i, 0))
    else:
        out_shape = (jax.ShapeDtypeStruct((NP, M), jnp.float32),
                     jax.ShapeDtypeStruct((NP, M // 2), jnp.int32))
        out_specs = (pl.BlockSpec((_BM, M), lambda i: (i, 0)),
                     pl.BlockSpec((_BM, M // 2), lambda i: (i, 0)))

    return pl.pallas_call(
        body,
        grid=(NP // _BM,),
        in_specs=[
            pl.BlockSpec((_BM, K), lambda i: (i, 0)),
            pl.BlockSpec((NC, _BM, K), lambda i: (0, i, 0)),
            pl.BlockSpec((K, M), lambda i: (0, 0)),
            pl.BlockSpec((1, M), lambda i: (0, 0)),
        ],
        out_specs=out_specs,
        out_shape=out_shape,
    )(x, parts, W, b)


def kernel(x, edge_index, W1, b1, W2, b2):
    x_p = jnp.pad(x, ((0, NP - N), (0, 0)))
    pad = jnp.full((EP - E,), N, jnp.int32)
    src2d = jnp.concatenate([edge_index[0], pad]).reshape(CHUNKS_TOTAL, CHUNK)
    dst2x = jnp.concatenate([edge_index[1], pad]).reshape(2 * CHUNKS_TOTAL, HC)

    packed_x = _tc_relu_pack(x_p)
    parts1 = _sc_scatter(packed_x, src2d, dst2x)
    h, packed_h = _tc_update(x_p, parts1, W1, b1.reshape(1, H), final=False)
    parts2 = _sc_scatter(packed_h, src2d, dst2x)
    out = _tc_update(h, parts2, W2, b2.reshape(1, C), final=True)
    return out[:N]


# speedup vs baseline: 1.5505x; 1.0239x over previous
import dataclasses
import functools

import jax
import jax.numpy as jnp
from jax import lax
from jax.experimental import pallas as pl
from jax.experimental.pallas import tpu as pltpu
from jax.experimental.pallas import tpu_sc as plsc

N, E, F, H, C = 10000, 320000, 128, 128, 64
NP = 10240
CHUNK = 64
EP = 327680
NC, NS = 2, 16
CHUNKS_TOTAL = EP // CHUNK
ROWS_PER_TILE = NP // NS
SECT = 8
CH0, CH1 = 192, 128
NC0_CHUNKS = NS * CH0
FH = F // 2


def _sc_scatter(table, src2d, dst2d):
    mesh = plsc.VectorSubcoreMesh(core_axis_name="c", subcore_axis_name="s")
    cp = pltpu.CompilerParams()
    for fld, val in (("needs_layout_passes", False),
                     ("use_tc_tiling_on_sc", False)):
        if fld in pltpu.CompilerParams.__dataclass_fields__:
            cp = dataclasses.replace(cp, **{fld: val})

    @functools.partial(
        pl.kernel,
        out_type=jax.ShapeDtypeStruct((NC, NP, F), jnp.float32),
        mesh=mesh,
        compiler_params=cp,
        scratch_types=[
            pltpu.VMEM_SHARED((NP, F), jnp.float32),
            pltpu.VMEM((2, SECT, CHUNK), jnp.int32),
            pltpu.VMEM((2, SECT, CHUNK), jnp.int32),
            pltpu.VMEM((2, CHUNK, FH), jnp.int32),
            pltpu.VMEM((2, CHUNK, F), jnp.float32),
        ] + [pltpu.SemaphoreType.DMA] * 6,
    )
    def k(table_hbm, src_hbm, dst_hbm, out_hbm,
          agg_sh, src_v, dst_v, ibuf, fbuf, g0, g1, s0, s1, isem, bsem):
        gsem = (g0, g1)
        ssem = (s0, s1)
        c = lax.axis_index("c")
        s = lax.axis_index("s")
        row0 = s * ROWS_PER_TILE

        def _idx_copies(sect0, t):
            return (
                pltpu.make_async_copy(src_hbm.at[pl.ds(sect0, SECT)],
                                      src_v.at[t], isem),
                pltpu.make_async_copy(dst_hbm.at[pl.ds(sect0, SECT)],
                                      dst_v.at[t], isem),
            )

        def _wait_gather(t, l, b):
            pltpu.make_async_copy(table_hbm.at[src_v.at[t, l]],
                                  ibuf.at[b], gsem[b]).wait()

        def _start_gather(t, l, b):
            pltpu.async_copy(table_hbm.at[src_v.at[t, l]], ibuf.at[b],
                             gsem[b])

        def _start_scatter(t, l, b):
            pltpu.async_copy(fbuf.at[b], agg_sh.at[dst_v.at[t, l]],
                             ssem[b], add=True)

        def _wait_scatter(t, l, b):
            pltpu.make_async_copy(fbuf.at[b], agg_sh.at[dst_v.at[t, l]],
                                  ssem[b]).wait()

        def _convert(b):
            @pl.loop(0, CHUNK, step=4)
            def _(r0):
                for dr in range(4):
                    r = r0 + dr
                    for g in range(FH // 16):
                        w = ibuf[b, r, pl.ds(g * 16, 16)]
                        fbuf[b, r, pl.ds(g * 16, 16)] = plsc.bitcast(
                            w << 16, jnp.float32)
                        fbuf[b, r, pl.ds(FH + g * 16, 16)] = plsc.bitcast(
                            w & jnp.int32(-65536), jnp.float32)

        def _run(nsect, base):
            @pl.loop(0, nsect)
            def _(sect):
                t = sect % 2
                nt = 1 - t
                not_first = sect > 0
                has_next = sect + 1 < nsect
                for l in range(SECT):
                    b = l % 2
                    _wait_gather(t, l, b)
                    if l < 2:
                        pl.when(not_first)(
                            lambda l=l, b=b: _wait_scatter(nt, SECT - 2 + l, b))
                    else:
                        _wait_scatter(t, l - 2, b)
                    _convert(b)
                    _start_scatter(t, l, b)
                    if l == 1:
                        pl.when(has_next)(
                            lambda: [cp.start() for cp in
                                     _idx_copies(base + (sect + 1) * SECT, nt)]
                            and None)
                    if l == SECT - 2:
                        pl.when(has_next)(
                            lambda: [cp.wait() for cp in
                                     _idx_copies(base + (sect + 1) * SECT, nt)]
                            and None)
                    if l < SECT - 2:
                        _start_gather(t, l + 2, b)
                    else:
                        pl.when(has_next)(
                            lambda l=l, b=b: _start_gather(nt, l - (SECT - 2), b))

            tl = (nsect - 1) % 2
            _wait_scatter(tl, SECT - 2, 0)
            _wait_scatter(tl, SECT - 1, 1)

        base0 = s * CH0
        base1 = NC0_CHUNKS + s * CH1
        base_c = jnp.where(c == 0, base0, base1)
        for cp in _idx_copies(base_c, 0):
            cp.start()

        @pl.loop(0, CHUNK)
        def _(i):
            for g in range(F // 16):
                fbuf[0, i, pl.ds(g * 16, 16)] = jnp.zeros((16,), jnp.float32)
        zcp = [
            pltpu.async_copy(fbuf.at[0],
                             agg_sh.at[pl.ds(row0 + t * CHUNK, CHUNK)], bsem)
            for t in range(ROWS_PER_TILE // CHUNK)
        ]
        for cp in zcp:
            cp.wait()
        for cp in _idx_copies(base_c, 0):
            cp.wait()
        _start_gather(0, 0, 0)
        _start_gather(0, 1, 1)
        plsc.subcore_barrier()

        nsect_c = jnp.where(c == 0, CH0 // SECT, CH1 // SECT)
        _run(nsect_c, base_c)
        plsc.subcore_barrier()

        wcp = []
        for t in range(ROWS_PER_TILE // CHUNK):
            sl = pl.ds(row0 + t * CHUNK, CHUNK)
            wcp.append(pltpu.async_copy(agg_sh.at[sl], out_hbm.at[c, sl], bsem))
        for cp in wcp:
            cp.wait()

    return k(table, src2d, dst2d)


_BM = 1024


def _pack_rows(z):
    lo = jax.lax.bitcast_convert_type(
        z[:, :FH].astype(jnp.bfloat16), jnp.uint16).astype(jnp.uint32)
    hi = jax.lax.bitcast_convert_type(
        z[:, FH:].astype(jnp.bfloat16), jnp.uint16).astype(jnp.uint32)
    return jax.lax.bitcast_convert_type(lo | (hi << 16), jnp.int32)


def _tc_relu_pack(x):
    def body(x_ref, o_ref):
        o_ref[...] = _pack_rows(jnp.maximum(x_ref[...], 0.0))

    return pl.pallas_call(
        body,
        grid=(NP // _BM,),
        in_specs=[pl.BlockSpec((_BM, F), lambda i: (i, 0))],
        out_specs=pl.BlockSpec((_BM, FH), lambda i: (i, 0)),
        out_shape=jax.ShapeDtypeStruct((NP, FH), jnp.int32),
    )(x)


def _tc_update(x, parts, W, b, final):
    K, M = W.shape

    def body(x_ref, p_ref, w_ref, b_ref, *o_refs):
        acc = x_ref[...] + p_ref[0] + p_ref[1]
        z = jax.lax.dot_general(
            acc, w_ref[...], (((1,), (0,)), ((), ())),
            precision=lax.Precision.HIGHEST,
            preferred_element_type=jnp.float32,
        ) + b_ref[...]
        if final:
            m = jnp.max(z, axis=1, keepdims=True)
            e = jnp.exp(z - m)
            o_refs[0][...] = (z - m) - jnp.log(jnp.sum(e, axis=1, keepdims=True))
        else:
            zr = jnp.maximum(z, 0.0)
            o_refs[0][...] = zr
            o_refs[1][...] = _pack_rows(zr)

    if final:
        out_shape = jax.ShapeDtypeStruct((NP, M), jnp.float32)
        out_specs = pl.BlockSpec((_BM, M), lambda i: (i, 0))
    else:
        out_shape = (jax.ShapeDtypeStruct((NP, M), jnp.float32),
                     jax.ShapeDtypeStruct((NP, M // 2), jnp.int32))
        out_specs = (pl.BlockSpec((_BM, M), lambda i: (i, 0)),
                     pl.BlockSpec((_BM, M // 2), lambda i: (i, 0)))

    return pl.pallas_call(
        body,
        grid=(NP // _BM,),
        in_specs=[
            pl.BlockSpec((_BM, K), lambda i: (i, 0)),
            pl.BlockSpec((NC, _BM, K), lambda i: (0, i, 0)),
            pl.BlockSpec((K, M), lambda i: (0, 0)),
            pl.BlockSpec((1, M), lambda i: (0, 0)),
        ],
        out_specs=out_specs,
        out_shape=out_shape,
    )(x, parts, W, b)


def kernel(x, edge_index, W1, b1, W2, b2):
    x_p = jnp.pad(x, ((0, NP - N), (0, 0)))
    pad = jnp.full((EP - E,), N, jnp.int32)
    src2d = jnp.concatenate([edge_index[0], pad]).reshape(CHUNKS_TOTAL, CHUNK)
    dst2d = jnp.concatenate([edge_index[1], pad]).reshape(CHUNKS_TOTAL, CHUNK)

    packed_x = _tc_relu_pack(x_p)
    parts1 = _sc_scatter(packed_x, src2d, dst2d)
    h, packed_h = _tc_update(x_p, parts1, W1, b1.reshape(1, H), final=False)
    parts2 = _sc_scatter(packed_h, src2d, dst2d)
    out = _tc_update(h, parts2, W2, b2.reshape(1, C), final=True)
    return out[:N]
